# baseline (speedup 1.0000x reference)
import functools

import jax
import jax.numpy as jnp
from jax import lax
from jax.experimental import pallas as pl
from jax.experimental.pallas import tpu as pltpu
from jax.experimental.pallas import tpu_sc as plsc

NUM_CODES = 512
CODE_DIM = 32
TOKENS = 4096
TOK_BLK = 256
NUM_BLOCKS = TOKENS // TOK_BLK
CODE_CHUNK = 128
NUM_CHUNKS = NUM_CODES // CODE_CHUNK

SC_CORES = 2
SC_SUBCORES = 16
SC_WORKERS = SC_CORES * SC_SUBCORES
TOK_PER_WORKER = TOKENS // SC_WORKERS


def _argmin_body(z_ref, cb_ref, idx_ref):
    z = z_ref[...]
    cb = cb_ref[...]
    c_sq = jnp.sum(cb * cb, axis=1)
    best_m = jnp.full((TOK_BLK, 1), jnp.inf, jnp.float32)
    best_i = jnp.zeros((TOK_BLK, 1), jnp.int32)
    iota = lax.broadcasted_iota(jnp.int32, (TOK_BLK, CODE_CHUNK), 1)
    for c in range(NUM_CHUNKS):
        cbc = cb[c * CODE_CHUNK:(c + 1) * CODE_CHUNK, :]
        dots = lax.dot_general(z, cbc, (((1,), (1,)), ((), ())),
                               precision=lax.Precision.HIGHEST,
                               preferred_element_type=jnp.float32)
        sc = c_sq[None, c * CODE_CHUNK:(c + 1) * CODE_CHUNK] - 2.0 * dots
        m_c = jnp.min(sc, axis=1, keepdims=True)
        i_c = jnp.min(jnp.where(sc == m_c, iota + c * CODE_CHUNK, NUM_CODES),
                      axis=1, keepdims=True)
        upd = m_c < best_m
        best_i = jnp.where(upd, i_c, best_i)
        best_m = jnp.where(upd, m_c, best_m)
    idx_ref[...] = best_i


_argmin_call = pl.pallas_call(
    _argmin_body,
    grid=(NUM_BLOCKS,),
    in_specs=[
        pl.BlockSpec((TOK_BLK, CODE_DIM), lambda i: (i, 0)),
        pl.BlockSpec((NUM_CODES, CODE_DIM), lambda i: (0, 0)),
    ],
    out_specs=pl.BlockSpec((TOK_BLK, 1), lambda i: (i, 0)),
    out_shape=jax.ShapeDtypeStruct((TOKENS, 1), jnp.int32),
)


@functools.lru_cache(maxsize=1)
def _make_sc_gather():
    mesh = plsc.VectorSubcoreMesh(core_axis_name="c", subcore_axis_name="s")

    @functools.partial(
        pl.kernel,
        mesh=mesh,
        out_type=jax.ShapeDtypeStruct((TOKENS, CODE_DIM), jnp.float32),
        scratch_types=[
            pltpu.VMEM((TOK_PER_WORKER,), jnp.int32),
            pltpu.VMEM((TOK_PER_WORKER, CODE_DIM), jnp.float32),
            pltpu.SemaphoreType.DMA,
        ],
        compiler_params=pltpu.CompilerParams(use_tc_tiling_on_sc=False),
    )
    def _sc_gather(table_hbm, idx_hbm, out_hbm, idx_v, rows_v, sem):
        wid = lax.axis_index("s") * SC_CORES + lax.axis_index("c")
        base = wid * TOK_PER_WORKER
        pltpu.sync_copy(idx_hbm.at[pl.ds(base, TOK_PER_WORKER)], idx_v)
        pltpu.async_copy(table_hbm.at[idx_v], rows_v, sem).wait()
        pltpu.sync_copy(rows_v, out_hbm.at[pl.ds(base, TOK_PER_WORKER)])

    return _sc_gather


def kernel(z_e, codebook):
    B, S, D = z_e.shape
    z = z_e.reshape(B * S, D)
    idx = _argmin_call(z, codebook).reshape(B * S)
    z_q = _make_sc_gather()(codebook, idx)
    return z_q.reshape(B, S, D), idx.reshape(B, S)


# speedup vs baseline: 1.0075x; 1.0075x over previous
import functools

import jax
import jax.numpy as jnp
from jax import lax
from jax.experimental import pallas as pl
from jax.experimental.pallas import tpu as pltpu
from jax.experimental.pallas import tpu_sc as plsc

NUM_CODES = 512
CODE_DIM = 32
TOKENS = 4096
TOK_BLK = 256
NUM_BLOCKS = TOKENS // TOK_BLK
CODE_CHUNK = 128
NUM_CHUNKS = NUM_CODES // CODE_CHUNK

SC_CORES = 2
SC_SUBCORES = 16
SC_WORKERS = SC_CORES * SC_SUBCORES
TOK_PER_WORKER = TOKENS // SC_WORKERS


def _argmin_body(z_ref, cb_ref, idx_ref):
    z = z_ref[...]
    cb = cb_ref[...]
    c_sq = jnp.sum(cb * cb, axis=1)
    best_m = jnp.full((TOK_BLK, 1), jnp.inf, jnp.float32)
    best_i = jnp.zeros((TOK_BLK, 1), jnp.int32)
    iota = lax.broadcasted_iota(jnp.int32, (TOK_BLK, CODE_CHUNK), 1)
    for c in range(NUM_CHUNKS):
        cbc = cb[c * CODE_CHUNK:(c + 1) * CODE_CHUNK, :]
        dots = lax.dot_general(z, cbc, (((1,), (1,)), ((), ())),
                               precision=lax.Precision.HIGHEST,
                               preferred_element_type=jnp.float32)
        sc = c_sq[None, c * CODE_CHUNK:(c + 1) * CODE_CHUNK] - 2.0 * dots
        m_c = jnp.min(sc, axis=1, keepdims=True)
        i_c = jnp.min(jnp.where(sc == m_c, iota + c * CODE_CHUNK, NUM_CODES),
                      axis=1, keepdims=True)
        upd = m_c < best_m
        best_i = jnp.where(upd, i_c, best_i)
        best_m = jnp.where(upd, m_c, best_m)
    idx_ref[...] = best_i


_argmin_call = pl.pallas_call(
    _argmin_body,
    grid=(NUM_BLOCKS,),
    in_specs=[
        pl.BlockSpec((TOK_BLK, CODE_DIM), lambda i: (i, 0)),
        pl.BlockSpec((NUM_CODES, CODE_DIM), lambda i: (0, 0)),
    ],
    out_specs=pl.BlockSpec((TOK_BLK, 1), lambda i: (i, 0)),
    out_shape=jax.ShapeDtypeStruct((TOKENS, 1), jnp.int32),
)


@functools.lru_cache(maxsize=1)
def _make_sc_gather():
    mesh = plsc.VectorSubcoreMesh(core_axis_name="c", subcore_axis_name="s")

    @functools.partial(
        pl.kernel,
        mesh=mesh,
        out_type=jax.ShapeDtypeStruct((TOKENS, CODE_DIM), jnp.float32),
        scratch_types=[
            pltpu.VMEM((TOK_PER_WORKER,), jnp.int32),
            pltpu.VMEM((TOK_PER_WORKER, CODE_DIM), jnp.float32),
            pltpu.SemaphoreType.DMA,
        ],
        compiler_params=pltpu.CompilerParams(use_tc_tiling_on_sc=False),
    )
    def _sc_gather(table_hbm, idx_hbm, out_hbm, idx_v, rows_v, sem):
        wid = lax.axis_index("s") * SC_CORES + lax.axis_index("c")
        base = wid * TOK_PER_WORKER
        pltpu.sync_copy(idx_hbm.at[pl.ds(base, TOK_PER_WORKER)], idx_v)
        pltpu.async_copy(table_hbm.at[idx_v], rows_v, sem).wait()
        pltpu.sync_copy(rows_v, out_hbm.at[pl.ds(base, TOK_PER_WORKER)])

    return _sc_gather


def kernel(z_e, codebook):
    B, S, D = z_e.shape
    z = z_e.reshape(B * S, D)
    idx = _argmin_call(z, codebook).reshape(B * S)
    z_q = codebook[idx]
    return z_q.reshape(B, S, D), idx.reshape(B, S)


# speedup vs baseline: 1.0109x; 1.0034x over previous
import functools

import jax
import jax.numpy as jnp
from jax import lax
from jax.experimental import pallas as pl
from jax.experimental.pallas import tpu as pltpu
from jax.experimental.pallas import tpu_sc as plsc

NUM_CODES = 512
CODE_DIM = 32
TOKENS = 4096
TOK_BLK = 256
NUM_BLOCKS = TOKENS // TOK_BLK
CODE_CHUNK = 128
NUM_CHUNKS = NUM_CODES // CODE_CHUNK

SC_CORES = 2
SC_SUBCORES = 16
SC_WORKERS = SC_CORES * SC_SUBCORES
TOK_PER_WORKER = TOKENS // SC_WORKERS


def _argmin_body(z_ref, cb_ref, idx_ref):
    z = z_ref[...]
    cb = cb_ref[...]
    c_sq = jnp.sum(cb * cb, axis=1)
    best_m = jnp.full((TOK_BLK, 1), jnp.inf, jnp.float32)
    best_i = jnp.zeros((TOK_BLK, 1), jnp.int32)
    iota = lax.broadcasted_iota(jnp.int32, (TOK_BLK, CODE_CHUNK), 1)
    for c in range(NUM_CHUNKS):
        cbc = cb[c * CODE_CHUNK:(c + 1) * CODE_CHUNK, :]
        dots = lax.dot_general(z, cbc, (((1,), (1,)), ((), ())),
                               precision=lax.Precision.DEFAULT,
                               preferred_element_type=jnp.float32)
        sc = c_sq[None, c * CODE_CHUNK:(c + 1) * CODE_CHUNK] - 2.0 * dots
        m_c = jnp.min(sc, axis=1, keepdims=True)
        i_c = jnp.min(jnp.where(sc == m_c, iota + c * CODE_CHUNK, NUM_CODES),
                      axis=1, keepdims=True)
        upd = m_c < best_m
        best_i = jnp.where(upd, i_c, best_i)
        best_m = jnp.where(upd, m_c, best_m)
    idx_ref[...] = best_i


_argmin_call = pl.pallas_call(
    _argmin_body,
    grid=(NUM_BLOCKS,),
    in_specs=[
        pl.BlockSpec((TOK_BLK, CODE_DIM), lambda i: (i, 0)),
        pl.BlockSpec((NUM_CODES, CODE_DIM), lambda i: (0, 0)),
    ],
    out_specs=pl.BlockSpec((TOK_BLK, 1), lambda i: (i, 0)),
    out_shape=jax.ShapeDtypeStruct((TOKENS, 1), jnp.int32),
)


@functools.lru_cache(maxsize=1)
def _make_sc_gather():
    mesh = plsc.VectorSubcoreMesh(core_axis_name="c", subcore_axis_name="s")

    @functools.partial(
        pl.kernel,
        mesh=mesh,
        out_type=jax.ShapeDtypeStruct((TOKENS, CODE_DIM), jnp.float32),
        scratch_types=[
            pltpu.VMEM((TOK_PER_WORKER,), jnp.int32),
            pltpu.VMEM((TOK_PER_WORKER, CODE_DIM), jnp.float32),
            pltpu.SemaphoreType.DMA,
        ],
        compiler_params=pltpu.CompilerParams(use_tc_tiling_on_sc=False),
    )
    def _sc_gather(table_hbm, idx_hbm, out_hbm, idx_v, rows_v, sem):
        wid = lax.axis_index("s") * SC_CORES + lax.axis_index("c")
        base = wid * TOK_PER_WORKER
        pltpu.sync_copy(idx_hbm.at[pl.ds(base, TOK_PER_WORKER)], idx_v)
        pltpu.async_copy(table_hbm.at[idx_v], rows_v, sem).wait()
        pltpu.sync_copy(rows_v, out_hbm.at[pl.ds(base, TOK_PER_WORKER)])

    return _sc_gather


def kernel(z_e, codebook):
    B, S, D = z_e.shape
    z = z_e.reshape(B * S, D)
    idx = _argmin_call(z, codebook).reshape(B * S)
    z_q = codebook[idx]
    return z_q.reshape(B, S, D), idx.reshape(B, S)


# speedup vs baseline: 33.4302x; 33.0698x over previous
import functools

import jax
import jax.numpy as jnp
from jax import lax
from jax.experimental import pallas as pl
from jax.experimental.pallas import tpu as pltpu
from jax.experimental.pallas import tpu_sc as plsc

NUM_CODES = 512
CODE_DIM = 32
TOKENS = 4096
TOK_BLK = 256
NUM_BLOCKS = TOKENS // TOK_BLK
CODE_CHUNK = 128
NUM_CHUNKS = NUM_CODES // CODE_CHUNK

SC_CORES = 2
SC_SUBCORES = 16
SC_WORKERS = SC_CORES * SC_SUBCORES
TOK_PER_WORKER = TOKENS // SC_WORKERS


def _argmin_body(z_ref, cb_ref, idx_ref):
    z = z_ref[...]
    cb = cb_ref[...]
    best_m = jnp.full((1, TOK_BLK), jnp.inf, jnp.float32)
    best_i = jnp.zeros((1, TOK_BLK), jnp.int32)
    iota = lax.broadcasted_iota(jnp.int32, (CODE_CHUNK, TOK_BLK), 0)
    for c in range(NUM_CHUNKS):
        cbc = cb[c * CODE_CHUNK:(c + 1) * CODE_CHUNK, :]
        c_sq = jnp.sum(cbc * cbc, axis=1, keepdims=True)
        dots = lax.dot_general(cbc, z, (((1,), (1,)), ((), ())),
                               precision=lax.Precision.HIGHEST,
                               preferred_element_type=jnp.float32)
        sc = c_sq - 2.0 * dots
        m_c = jnp.min(sc, axis=0, keepdims=True)
        i_c = jnp.min(jnp.where(sc == m_c, iota + c * CODE_CHUNK, NUM_CODES),
                      axis=0, keepdims=True)
        upd = m_c < best_m
        best_i = jnp.where(upd, i_c, best_i)
        best_m = jnp.where(upd, m_c, best_m)
    idx_ref[...] = best_i[None]


_argmin_call = pl.pallas_call(
    _argmin_body,
    grid=(NUM_BLOCKS,),
    in_specs=[
        pl.BlockSpec((TOK_BLK, CODE_DIM), lambda i: (i, 0)),
        pl.BlockSpec((NUM_CODES, CODE_DIM), lambda i: (0, 0)),
    ],
    out_specs=pl.BlockSpec((1, 1, TOK_BLK), lambda i: (i, 0, 0)),
    out_shape=jax.ShapeDtypeStruct((NUM_BLOCKS, 1, TOK_BLK), jnp.int32),
)


@functools.lru_cache(maxsize=1)
def _make_sc_gather():
    mesh = plsc.VectorSubcoreMesh(core_axis_name="c", subcore_axis_name="s")

    @functools.partial(
        pl.kernel,
        mesh=mesh,
        out_type=jax.ShapeDtypeStruct((TOKENS, CODE_DIM), jnp.float32),
        scratch_types=[
            pltpu.VMEM((TOK_PER_WORKER,), jnp.int32),
            pltpu.VMEM((TOK_PER_WORKER, CODE_DIM), jnp.float32),
            pltpu.SemaphoreType.DMA,
        ],
        compiler_params=pltpu.CompilerParams(use_tc_tiling_on_sc=False),
    )
    def _sc_gather(table_hbm, idx_hbm, out_hbm, idx_v, rows_v, sem):
        wid = lax.axis_index("s") * SC_CORES + lax.axis_index("c")
        base = wid * TOK_PER_WORKER
        pltpu.sync_copy(idx_hbm.at[pl.ds(base, TOK_PER_WORKER)], idx_v)
        pltpu.async_copy(table_hbm.at[idx_v], rows_v, sem).wait()
        pltpu.sync_copy(rows_v, out_hbm.at[pl.ds(base, TOK_PER_WORKER)])

    return _sc_gather


def kernel(z_e, codebook):
    B, S, D = z_e.shape
    z = z_e.reshape(B * S, D)
    idx = _argmin_call(z, codebook).reshape(B * S)
    z_q = codebook[idx]
    return z_q.reshape(B, S, D), idx.reshape(B, S)
